# baseline (speedup 1.0000x reference)
import functools

import jax
import jax.numpy as jnp
from jax import lax
from jax.experimental import pallas as pl
from jax.experimental.pallas import tpu as pltpu
from jax.experimental.pallas import tpu_sc as plsc

N_NODES = 10000
N_EDGES = 320000
D_FEAT = 128
D_HID = 256
D_OUT = 256

NC = 2
NS = 16
NW = NC * NS
CHUNK = 128
CPT = 80
EPT = CPT * CHUNK
E_PAD = NW * EPT
ROWS_PAD = 10240
RPT = ROWS_PAD // NS


def _sc_body(x_hbm, src_hbm, dst_hbm, zeros_hbm, zeros1_hbm,
             aggp_hbm, degp_hbm,
             src_v, dst_v, rows_v0, rows_v1, ones_v, agg_sh, deg_sh,
             sem_g0, sem_g1, sem_d):
    c = lax.axis_index("c")
    s = lax.axis_index("s")
    wid = s * NC + c

    pltpu.sync_copy(zeros_hbm, agg_sh.at[pl.ds(s * RPT, RPT)])
    pltpu.sync_copy(zeros1_hbm.at[pl.ds(s * RPT, RPT)],
                    deg_sh.at[pl.ds(s * RPT, RPT)])

    ones = jnp.ones((16,), jnp.float32)
    for k in range(CHUNK // 16):
        ones_v[pl.ds(k * 16, 16)] = ones

    base = wid * EPT
    pltpu.sync_copy(src_hbm.at[pl.ds(base, EPT)], src_v)
    pltpu.sync_copy(dst_hbm.at[pl.ds(wid * CPT, CPT)], dst_v)

    plsc.subcore_barrier()

    rows = (rows_v0, rows_v1)
    sem_g = (sem_g0, sem_g1)

    def gather(i, b):
        return pltpu.make_async_copy(
            x_hbm.at[src_v.at[pl.ds(i * CHUNK, CHUNK)]], rows[b], sem_g[b])

    def step(i, carry):
        g = gather(i, 0)
        g.start()
        g.wait()
        pltpu.sync_copy(rows[0], agg_sh.at[dst_v.at[i]], add=True)
        pltpu.sync_copy(ones_v, deg_sh.at[dst_v.at[i]], add=True)
        return carry

    lax.fori_loop(0, CPT, step, 0)

    plsc.subcore_barrier()

    pltpu.sync_copy(agg_sh.at[pl.ds(s * RPT, RPT)],
                    aggp_hbm.at[c].at[pl.ds(s * RPT, RPT)])
    pltpu.sync_copy(deg_sh.at[pl.ds(s * RPT, RPT)],
                    degp_hbm.at[c].at[pl.ds(s * RPT, RPT)])


def _mlp_body(a0, a1, d0, d1, w1, b1, w2, b2, out):
    a = a0[0] + a1[0]
    d = d0[0] + d1[0]
    a = a / jnp.maximum(d, 1.0)
    h = jnp.dot(a, w1[...], preferred_element_type=jnp.float32) + b1[...]
    h = jnp.maximum(h, 0.0)
    out[...] = jnp.dot(h, w2[...], preferred_element_type=jnp.float32) + b2[...]


def kernel(x, edge_index, W1, b1, W2, b2):
    src = edge_index[0].astype(jnp.int32)
    dst = edge_index[1].astype(jnp.int32)
    pad = E_PAD - N_EDGES
    src = jnp.concatenate([src, jnp.zeros((pad,), jnp.int32)])
    dst_fill = N_NODES + jnp.arange(pad, dtype=jnp.int32) % (ROWS_PAD - N_NODES)
    dst = jnp.concatenate([dst, dst_fill]).reshape(E_PAD // CHUNK, CHUNK)
    zeros = jnp.zeros((RPT, D_FEAT), jnp.float32)
    zeros1 = jnp.zeros((ROWS_PAD,), jnp.float32)

    mesh = plsc.VectorSubcoreMesh(core_axis_name="c", subcore_axis_name="s",
                                  num_cores=NC, num_subcores=NS)
    sc = pl.kernel(
        _sc_body,
        out_type=(
            jax.ShapeDtypeStruct((NC, ROWS_PAD, D_FEAT), jnp.float32),
            jax.ShapeDtypeStruct((NC, ROWS_PAD), jnp.float32),
        ),
        mesh=mesh,
        scratch_types=[
            pltpu.VMEM((EPT,), jnp.int32),
            pltpu.VMEM((CPT, CHUNK), jnp.int32),
            pltpu.VMEM((CHUNK, D_FEAT), jnp.float32),
            pltpu.VMEM((CHUNK, D_FEAT), jnp.float32),
            pltpu.VMEM((CHUNK,), jnp.float32),
            pltpu.VMEM_SHARED((ROWS_PAD, D_FEAT), jnp.float32),
            pltpu.VMEM_SHARED((ROWS_PAD,), jnp.float32),
            pltpu.SemaphoreType.DMA,
            pltpu.SemaphoreType.DMA,
            pltpu.SemaphoreType.DMA,
        ],
    )
    aggp, degp = sc(x, src, dst, zeros, zeros1)
    degp = degp.reshape(NC, ROWS_PAD, 1)

    R = 400
    grid = (N_NODES // R,)
    out = pl.pallas_call(
        _mlp_body,
        grid=grid,
        in_specs=[
            pl.BlockSpec((1, R, D_FEAT), lambda i: (0, i, 0)),
            pl.BlockSpec((1, R, D_FEAT), lambda i: (1, i, 0)),
            pl.BlockSpec((1, R, 1), lambda i: (0, i, 0)),
            pl.BlockSpec((1, R, 1), lambda i: (1, i, 0)),
            pl.BlockSpec((D_FEAT, D_HID), lambda i: (0, 0)),
            pl.BlockSpec((1, D_HID), lambda i: (0, 0)),
            pl.BlockSpec((D_HID, D_OUT), lambda i: (0, 0)),
            pl.BlockSpec((1, D_OUT), lambda i: (0, 0)),
        ],
        out_specs=pl.BlockSpec((R, D_OUT), lambda i: (i, 0)),
        out_shape=jax.ShapeDtypeStruct((N_NODES, D_OUT), jnp.float32),
    )(aggp, aggp, degp, degp, W1, b1.reshape(1, D_HID), W2,
      b2.reshape(1, D_OUT))
    return out


# speedup vs baseline: 2.3569x; 2.3569x over previous
import functools

import jax
import jax.numpy as jnp
from jax import lax
from jax.experimental import pallas as pl
from jax.experimental.pallas import tpu as pltpu
from jax.experimental.pallas import tpu_sc as plsc

N_NODES = 10000
N_EDGES = 320000
D_FEAT = 128
D_HID = 256
D_OUT = 256

NC = 2
NS = 16
NW = NC * NS
CHUNK = 128
CPT = 80
EPT = CPT * CHUNK
E_PAD = NW * EPT
ROWS_PAD = 10240
RPT = ROWS_PAD // NS


def _sc_body(x_hbm, src_hbm, dst_hbm, zeros_hbm, zeros1_hbm,
             aggp_hbm, degp_hbm,
             src_v, dst_v, rows_v0, rows_v1, ones_v, agg_sh, deg_sh,
             sem_g0, sem_g1, sem_d):
    c = lax.axis_index("c")
    s = lax.axis_index("s")
    wid = s * NC + c

    pltpu.sync_copy(zeros_hbm, agg_sh.at[pl.ds(s * RPT, RPT)])
    pltpu.sync_copy(zeros1_hbm.at[pl.ds(s * RPT, RPT)],
                    deg_sh.at[pl.ds(s * RPT, RPT)])

    ones = jnp.ones((16,), jnp.float32)
    for k in range(CHUNK // 16):
        ones_v[pl.ds(k * 16, 16)] = ones

    base = wid * EPT
    pltpu.sync_copy(src_hbm.at[pl.ds(base, EPT)], src_v)
    pltpu.sync_copy(dst_hbm.at[pl.ds(wid * CPT, CPT)], dst_v)

    plsc.subcore_barrier()

    rows = (rows_v0, rows_v1)
    sem_g = (sem_g0, sem_g1)

    def gather(i, b):
        return pltpu.make_async_copy(
            x_hbm.at[src_v.at[pl.ds(i * CHUNK, CHUNK)]], rows[b], sem_g[b])

    def step(i, carry):
        g = gather(i, 0)
        g.start()
        g.wait()
        pltpu.sync_copy(rows[0], agg_sh.at[dst_v.at[i]], add=True)
        pltpu.sync_copy(ones_v, deg_sh.at[dst_v.at[i]], add=True)
        return carry

    lax.fori_loop(0, CPT, step, 0)

    plsc.subcore_barrier()

    pltpu.sync_copy(agg_sh.at[pl.ds(s * RPT, RPT)],
                    aggp_hbm.at[c].at[pl.ds(s * RPT, RPT)])
    pltpu.sync_copy(deg_sh.at[pl.ds(s * RPT, RPT)],
                    degp_hbm.at[c].at[pl.ds(s * RPT, RPT)])


def _mlp_body(a0, a1, d0, d1, w1, b1, w2, b2, out):
    a = a0[0] + a1[0]
    d = d0[0] + d1[0]
    a = a / jnp.maximum(d, 1.0)
    h = jnp.dot(a, w1[...], preferred_element_type=jnp.float32) + b1[...]
    h = jnp.maximum(h, 0.0)
    out[...] = jnp.dot(h, w2[...], preferred_element_type=jnp.float32) + b2[...]


def kernel(x, edge_index, W1, b1, W2, b2):
    src = edge_index[0].astype(jnp.int32)
    dst = edge_index[1].astype(jnp.int32)
    pad = E_PAD - N_EDGES
    src_fill = jnp.arange(pad, dtype=jnp.int32) * 37 % N_NODES
    src = jnp.concatenate([src, src_fill])
    dst_fill = N_NODES + jnp.arange(pad, dtype=jnp.int32) % (ROWS_PAD - N_NODES)
    dst = jnp.concatenate([dst, dst_fill]).reshape(E_PAD // CHUNK, CHUNK)
    zeros = jnp.zeros((RPT, D_FEAT), jnp.float32)
    zeros1 = jnp.zeros((ROWS_PAD,), jnp.float32)

    mesh = plsc.VectorSubcoreMesh(core_axis_name="c", subcore_axis_name="s",
                                  num_cores=NC, num_subcores=NS)
    sc = pl.kernel(
        _sc_body,
        out_type=(
            jax.ShapeDtypeStruct((NC, ROWS_PAD, D_FEAT), jnp.float32),
            jax.ShapeDtypeStruct((NC, ROWS_PAD), jnp.float32),
        ),
        mesh=mesh,
        scratch_types=[
            pltpu.VMEM((EPT,), jnp.int32),
            pltpu.VMEM((CPT, CHUNK), jnp.int32),
            pltpu.VMEM((CHUNK, D_FEAT), jnp.float32),
            pltpu.VMEM((CHUNK, D_FEAT), jnp.float32),
            pltpu.VMEM((CHUNK,), jnp.float32),
            pltpu.VMEM_SHARED((ROWS_PAD, D_FEAT), jnp.float32),
            pltpu.VMEM_SHARED((ROWS_PAD,), jnp.float32),
            pltpu.SemaphoreType.DMA,
            pltpu.SemaphoreType.DMA,
            pltpu.SemaphoreType.DMA,
        ],
    )
    aggp, degp = sc(x, src, dst, zeros, zeros1)
    degp = degp.reshape(NC, ROWS_PAD, 1)

    R = 400
    grid = (N_NODES // R,)
    out = pl.pallas_call(
        _mlp_body,
        grid=grid,
        in_specs=[
            pl.BlockSpec((1, R, D_FEAT), lambda i: (0, i, 0)),
            pl.BlockSpec((1, R, D_FEAT), lambda i: (1, i, 0)),
            pl.BlockSpec((1, R, 1), lambda i: (0, i, 0)),
            pl.BlockSpec((1, R, 1), lambda i: (1, i, 0)),
            pl.BlockSpec((D_FEAT, D_HID), lambda i: (0, 0)),
            pl.BlockSpec((1, D_HID), lambda i: (0, 0)),
            pl.BlockSpec((D_HID, D_OUT), lambda i: (0, 0)),
            pl.BlockSpec((1, D_OUT), lambda i: (0, 0)),
        ],
        out_specs=pl.BlockSpec((R, D_OUT), lambda i: (i, 0)),
        out_shape=jax.ShapeDtypeStruct((N_NODES, D_OUT), jnp.float32),
    )(aggp, aggp, degp, degp, W1, b1.reshape(1, D_HID), W2,
      b2.reshape(1, D_OUT))
    return out


# speedup vs baseline: 2.9720x; 1.2610x over previous
import functools

import jax
import jax.numpy as jnp
from jax import lax
from jax.experimental import pallas as pl
from jax.experimental.pallas import tpu as pltpu
from jax.experimental.pallas import tpu_sc as plsc

N_NODES = 10000
N_EDGES = 320000
D_FEAT = 128
D_HID = 256
D_OUT = 256

NC = 2
NS = 16
NW = NC * NS
CHUNK = 64
CPT = 160
EPT = CPT * CHUNK
E_PAD = NW * EPT
ROWS_PAD = 10240
RPT = ROWS_PAD // NS


def _sc_body(x_hbm, src_hbm, dst_hbm, zeros_hbm, zeros1_hbm,
             aggp_hbm, degp_hbm,
             src_v, dst_v, rows_v0, rows_v1, ones_v, agg_sh, deg_sh,
             sem_g0, sem_g1, sem_d):
    c = lax.axis_index("c")
    s = lax.axis_index("s")
    wid = s * NC + c

    pltpu.sync_copy(zeros_hbm, agg_sh.at[pl.ds(s * RPT, RPT)])
    pltpu.sync_copy(zeros1_hbm.at[pl.ds(s * RPT, RPT)],
                    deg_sh.at[pl.ds(s * RPT, RPT)])

    ones = jnp.ones((16,), jnp.float32)
    for k in range(CHUNK // 16):
        ones_v[pl.ds(k * 16, 16)] = ones

    base = wid * EPT
    pltpu.sync_copy(src_hbm.at[pl.ds(base, EPT)], src_v)
    pltpu.sync_copy(dst_hbm.at[pl.ds(wid * CPT, CPT)], dst_v)

    plsc.subcore_barrier()

    rows = (rows_v0, rows_v1)
    sem_g = (sem_g0, sem_g1)

    def gather(i, b):
        return pltpu.make_async_copy(
            x_hbm.at[src_v.at[pl.ds(i * CHUNK, CHUNK)]], rows[b], sem_g[b])

    gather(0, 0).start()

    def step(k, carry):
        for b in range(2):
            i = k * 2 + b
            nxt = i + 1

            @pl.when(nxt < CPT)
            def _():
                gather(nxt, (b + 1) % 2).start()

            gather(i, b).wait()
            deg_cp = pltpu.async_copy(ones_v, deg_sh.at[dst_v.at[i]], sem_d,
                                      add=True)
            pltpu.sync_copy(rows[b], agg_sh.at[dst_v.at[i]], add=True)
            deg_cp.wait()
        return carry

    lax.fori_loop(0, CPT // 2, step, 0)

    plsc.subcore_barrier()

    pltpu.sync_copy(agg_sh.at[pl.ds(s * RPT, RPT)],
                    aggp_hbm.at[c].at[pl.ds(s * RPT, RPT)])
    pltpu.sync_copy(deg_sh.at[pl.ds(s * RPT, RPT)],
                    degp_hbm.at[c].at[pl.ds(s * RPT, RPT)])


def _mlp_body(a0, a1, d0, d1, w1, b1, w2, b2, out):
    a = a0[0] + a1[0]
    d = d0[0] + d1[0]
    a = a / jnp.maximum(d, 1.0)
    h = jnp.dot(a, w1[...], preferred_element_type=jnp.float32) + b1[...]
    h = jnp.maximum(h, 0.0)
    out[...] = jnp.dot(h, w2[...], preferred_element_type=jnp.float32) + b2[...]


def kernel(x, edge_index, W1, b1, W2, b2):
    src = edge_index[0].astype(jnp.int32)
    dst = edge_index[1].astype(jnp.int32)
    pad = E_PAD - N_EDGES
    src_fill = jnp.arange(pad, dtype=jnp.int32) * 37 % N_NODES
    src = jnp.concatenate([src, src_fill])
    dst_fill = N_NODES + jnp.arange(pad, dtype=jnp.int32) % (ROWS_PAD - N_NODES)
    dst = jnp.concatenate([dst, dst_fill]).reshape(E_PAD // CHUNK, CHUNK)
    zeros = jnp.zeros((RPT, D_FEAT), jnp.float32)
    zeros1 = jnp.zeros((ROWS_PAD,), jnp.float32)

    mesh = plsc.VectorSubcoreMesh(core_axis_name="c", subcore_axis_name="s",
                                  num_cores=NC, num_subcores=NS)
    sc = pl.kernel(
        _sc_body,
        out_type=(
            jax.ShapeDtypeStruct((NC, ROWS_PAD, D_FEAT), jnp.float32),
            jax.ShapeDtypeStruct((NC, ROWS_PAD), jnp.float32),
        ),
        mesh=mesh,
        scratch_types=[
            pltpu.VMEM((EPT,), jnp.int32),
            pltpu.VMEM((CPT, CHUNK), jnp.int32),
            pltpu.VMEM((CHUNK, D_FEAT), jnp.float32),
            pltpu.VMEM((CHUNK, D_FEAT), jnp.float32),
            pltpu.VMEM((CHUNK,), jnp.float32),
            pltpu.VMEM_SHARED((ROWS_PAD, D_FEAT), jnp.float32),
            pltpu.VMEM_SHARED((ROWS_PAD,), jnp.float32),
            pltpu.SemaphoreType.DMA,
            pltpu.SemaphoreType.DMA,
            pltpu.SemaphoreType.DMA,
        ],
    )
    aggp, degp = sc(x, src, dst, zeros, zeros1)
    degp = degp.reshape(NC, ROWS_PAD, 1)

    R = 400
    grid = (N_NODES // R,)
    out = pl.pallas_call(
        _mlp_body,
        grid=grid,
        in_specs=[
            pl.BlockSpec((1, R, D_FEAT), lambda i: (0, i, 0)),
            pl.BlockSpec((1, R, D_FEAT), lambda i: (1, i, 0)),
            pl.BlockSpec((1, R, 1), lambda i: (0, i, 0)),
            pl.BlockSpec((1, R, 1), lambda i: (1, i, 0)),
            pl.BlockSpec((D_FEAT, D_HID), lambda i: (0, 0)),
            pl.BlockSpec((1, D_HID), lambda i: (0, 0)),
            pl.BlockSpec((D_HID, D_OUT), lambda i: (0, 0)),
            pl.BlockSpec((1, D_OUT), lambda i: (0, 0)),
        ],
        out_specs=pl.BlockSpec((R, D_OUT), lambda i: (i, 0)),
        out_shape=jax.ShapeDtypeStruct((N_NODES, D_OUT), jnp.float32),
    )(aggp, aggp, degp, degp, W1, b1.reshape(1, D_HID), W2,
      b2.reshape(1, D_OUT))
    return out


# speedup vs baseline: 3.1273x; 1.0523x over previous
import functools

import jax
import jax.numpy as jnp
from jax import lax
from jax.experimental import pallas as pl
from jax.experimental.pallas import tpu as pltpu
from jax.experimental.pallas import tpu_sc as plsc

N_NODES = 10000
N_EDGES = 320000
D_FEAT = 128
D_HID = 256
D_OUT = 256

NC = 2
NS = 16
NW = NC * NS
CHUNK = 64
CPT = 168
EPT = CPT * CHUNK
E_PAD = NW * EPT
ROWS_PAD = 10240
RPT = ROWS_PAD // NS


def _sc_body(x_hbm, src_hbm, dst_hbm, zeros_hbm, zeros1_hbm,
             aggp_hbm, degp_hbm,
             src_v, dst_v, rows_v0, rows_v1, rows_v2, ones_v, agg_sh, deg_sh,
             sem_g0, sem_g1, sem_g2, sem_s0, sem_s1, sem_s2, sem_d):
    c = lax.axis_index("c")
    s = lax.axis_index("s")
    wid = s * NC + c

    pltpu.sync_copy(zeros_hbm, agg_sh.at[pl.ds(s * RPT, RPT)])
    pltpu.sync_copy(zeros1_hbm.at[pl.ds(s * RPT, RPT)],
                    deg_sh.at[pl.ds(s * RPT, RPT)])

    ones = jnp.ones((16,), jnp.float32)
    for k in range(CHUNK // 16):
        ones_v[pl.ds(k * 16, 16)] = ones

    rows = (rows_v0, rows_v1, rows_v2)
    sem_g = (sem_g0, sem_g1, sem_g2)
    sem_s = (sem_s0, sem_s1, sem_s2)
    HALF = CPT // 2
    base = wid * EPT

    def gather(i, b):
        return pltpu.make_async_copy(
            x_hbm.at[src_v.at[pl.ds(i * CHUNK, CHUNK)]], rows[b], sem_g[b])

    def scat_drain(b, i):
        pltpu.make_async_copy(rows[b], agg_sh.at[dst_v.at[i]],
                              sem_s[b]).wait()

    plsc.subcore_barrier()

    for h in range(2):
        pltpu.sync_copy(src_hbm.at[pl.ds(base + h * HALF * CHUNK,
                                         HALF * CHUNK)], src_v)
        pltpu.sync_copy(dst_hbm.at[wid, h], dst_v)

        gather(0, 0).start()
        gather(1, 1).start()

        def step(k, carry):
            for b in range(3):
                i = k * 3 + b
                gather(i, b).wait()
                pltpu.async_copy(rows[b], agg_sh.at[dst_v.at[i]], sem_s[b],
                                 add=True)
                pltpu.async_copy(ones_v, deg_sh.at[dst_v.at[i]], sem_d,
                                 add=True)
                pltpu.make_async_copy(ones_v, deg_sh.at[dst_v.at[i]],
                                      sem_d).wait()
                nxt = i + 2
                slot = (b + 2) % 3

                @pl.when(nxt < HALF)
                def _():
                    @pl.when(i > 0)
                    def _():
                        scat_drain(slot, i)
                    gather(nxt, slot).start()
            return carry

        lax.fori_loop(0, HALF // 3, step, 0)

        for b in range(3):
            scat_drain(b, HALF - 1)

    plsc.subcore_barrier()

    pltpu.sync_copy(agg_sh.at[pl.ds(s * RPT, RPT)],
                    aggp_hbm.at[c].at[pl.ds(s * RPT, RPT)])
    pltpu.sync_copy(deg_sh.at[pl.ds(s * RPT, RPT)],
                    degp_hbm.at[c].at[pl.ds(s * RPT, RPT)])


def _mlp_body(a0, a1, d0, d1, w1, b1, w2, b2, out):
    a = a0[0] + a1[0]
    d = d0[0] + d1[0]
    a = a / jnp.maximum(d, 1.0)
    h = jnp.dot(a, w1[...], preferred_element_type=jnp.float32) + b1[...]
    h = jnp.maximum(h, 0.0)
    out[...] = jnp.dot(h, w2[...], preferred_element_type=jnp.float32) + b2[...]


def kernel(x, edge_index, W1, b1, W2, b2):
    src = edge_index[0].astype(jnp.int32)
    dst = edge_index[1].astype(jnp.int32)
    pad = E_PAD - N_EDGES
    src_fill = jnp.arange(pad, dtype=jnp.int32) * 37 % N_NODES
    src = jnp.concatenate([src, src_fill])
    dst_fill = N_NODES + jnp.arange(pad, dtype=jnp.int32) % (ROWS_PAD - N_NODES)
    dst = jnp.concatenate([dst, dst_fill]).reshape(NW, 2, CPT // 2, CHUNK)
    zeros = jnp.zeros((RPT, D_FEAT), jnp.float32)
    zeros1 = jnp.zeros((ROWS_PAD,), jnp.float32)

    mesh = plsc.VectorSubcoreMesh(core_axis_name="c", subcore_axis_name="s",
                                  num_cores=NC, num_subcores=NS)
    sc = pl.kernel(
        _sc_body,
        out_type=(
            jax.ShapeDtypeStruct((NC, ROWS_PAD, D_FEAT), jnp.float32),
            jax.ShapeDtypeStruct((NC, ROWS_PAD), jnp.float32),
        ),
        mesh=mesh,
        scratch_types=[
            pltpu.VMEM((EPT // 2,), jnp.int32),
            pltpu.VMEM((CPT // 2, CHUNK), jnp.int32),
            pltpu.VMEM((CHUNK, D_FEAT), jnp.float32),
            pltpu.VMEM((CHUNK, D_FEAT), jnp.float32),
            pltpu.VMEM((CHUNK, D_FEAT), jnp.float32),
            pltpu.VMEM((CHUNK,), jnp.float32),
            pltpu.VMEM_SHARED((ROWS_PAD, D_FEAT), jnp.float32),
            pltpu.VMEM_SHARED((ROWS_PAD,), jnp.float32),
        ] + [pltpu.SemaphoreType.DMA] * 7,
    )
    aggp, degp = sc(x, src, dst, zeros, zeros1)
    degp = degp.reshape(NC, ROWS_PAD, 1)

    R = 400
    grid = (N_NODES // R,)
    out = pl.pallas_call(
        _mlp_body,
        grid=grid,
        in_specs=[
            pl.BlockSpec((1, R, D_FEAT), lambda i: (0, i, 0)),
            pl.BlockSpec((1, R, D_FEAT), lambda i: (1, i, 0)),
            pl.BlockSpec((1, R, 1), lambda i: (0, i, 0)),
            pl.BlockSpec((1, R, 1), lambda i: (1, i, 0)),
            pl.BlockSpec((D_FEAT, D_HID), lambda i: (0, 0)),
            pl.BlockSpec((1, D_HID), lambda i: (0, 0)),
            pl.BlockSpec((D_HID, D_OUT), lambda i: (0, 0)),
            pl.BlockSpec((1, D_OUT), lambda i: (0, 0)),
        ],
        out_specs=pl.BlockSpec((R, D_OUT), lambda i: (i, 0)),
        out_shape=jax.ShapeDtypeStruct((N_NODES, D_OUT), jnp.float32),
    )(aggp, aggp, degp, degp, W1, b1.reshape(1, D_HID), W2,
      b2.reshape(1, D_OUT))
    return out


# speedup vs baseline: 3.3182x; 1.0610x over previous
import jax
import jax.numpy as jnp
from jax import lax
from jax.experimental import pallas as pl
from jax.experimental.pallas import tpu as pltpu
from jax.experimental.pallas import tpu_sc as plsc

N_NODES = 10000
N_EDGES = 320000
D_FEAT = 128
D_HID = 256
D_OUT = 256

NC = 2
NS = 16
NW = NC * NS
CHUNK = 64
CPT = 168
EPT = CPT * CHUNK
E_PAD = NW * EPT
HALF = CPT // 2
EPH = HALF * CHUNK
ROWS_PAD = 10240
RPT = ROWS_PAD // NS


PREP_B = 6144


def _prep_body(ei, src_out, dst_out):
    g = pl.program_id(0)
    gi = g * PREP_B + jax.lax.broadcasted_iota(jnp.int32, (1, PREP_B), 1)
    real = gi < N_EDGES
    src = jnp.where(real, ei[0:1, :], gi & 8191)
    dst = jnp.where(real, ei[1:2, :], N_NODES + (gi & 127))
    src_out[...] = src.reshape(PREP_B)
    dst_out[...] = dst.reshape(PREP_B)


def _sc_body(x_hbm, src_hbm, dst_hbm, zeros_hbm, zeros1_hbm,
             aggp_hbm, degp_hbm,
             src_v, dst_v, db0, db1, db2, rows_v0, rows_v1, rows_v2, ones_v,
             agg_sh, deg_sh,
             sem_g0, sem_g1, sem_g2, sem_s0, sem_s1, sem_s2,
             sem_d0, sem_d1, sem_d2):
    c = lax.axis_index("c")
    s = lax.axis_index("s")
    wid = s * NC + c

    pltpu.sync_copy(zeros_hbm, agg_sh.at[pl.ds(s * RPT, RPT)])
    pltpu.sync_copy(zeros1_hbm.at[pl.ds(s * RPT, RPT)],
                    deg_sh.at[pl.ds(s * RPT, RPT)])

    ones = jnp.ones((16,), jnp.float32)
    for k in range(CHUNK // 16):
        ones_v[pl.ds(k * 16, 16)] = ones

    rows = (rows_v0, rows_v1, rows_v2)
    dstbuf = (db0, db1, db2)
    sem_g = (sem_g0, sem_g1, sem_g2)
    sem_s = (sem_s0, sem_s1, sem_s2)
    sem_d = (sem_d0, sem_d1, sem_d2)
    base = wid * EPT

    def gather(i, b):
        return pltpu.make_async_copy(
            x_hbm.at[src_v.at[pl.ds(i * CHUNK, CHUNK)]], rows[b], sem_g[b])

    def stage_dst(i, b):
        for j in range(CHUNK // 16):
            dstbuf[b][pl.ds(j * 16, 16)] = dst_v[pl.ds(i * CHUNK + j * 16, 16)]

    def drains(b):
        pltpu.make_async_copy(rows[b], agg_sh.at[dstbuf[b]], sem_s[b]).wait()
        pltpu.make_async_copy(ones_v, deg_sh.at[dstbuf[b]], sem_d[b]).wait()

    plsc.subcore_barrier()

    for h in range(2):
        pltpu.sync_copy(src_hbm.at[pl.ds(base + h * EPH, EPH)], src_v)
        pltpu.sync_copy(dst_hbm.at[pl.ds(base + h * EPH, EPH)], dst_v)

        stage_dst(0, 0)
        stage_dst(1, 1)
        gather(0, 0).start()
        gather(1, 1).start()

        def step(k, carry):
            for b in range(3):
                i = k * 3 + b
                gather(i, b).wait()
                pltpu.async_copy(rows[b], agg_sh.at[dstbuf[b]], sem_s[b],
                                 add=True)
                pltpu.async_copy(ones_v, deg_sh.at[dstbuf[b]], sem_d[b],
                                 add=True)
                nxt = i + 2
                slot = (b + 2) % 3

                @pl.when(nxt < HALF)
                def _():
                    @pl.when(i > 0)
                    def _():
                        drains(slot)
                    stage_dst(nxt, slot)
                    gather(nxt, slot).start()
            return carry

        lax.fori_loop(0, HALF // 3, step, 0)

        for b in range(3):
            drains(b)

    plsc.subcore_barrier()

    pltpu.sync_copy(agg_sh.at[pl.ds(s * RPT, RPT)],
                    aggp_hbm.at[c].at[pl.ds(s * RPT, RPT)])
    pltpu.sync_copy(deg_sh.at[pl.ds(s * RPT, RPT)],
                    degp_hbm.at[c].at[pl.ds(s * RPT, RPT)])


def _mlp_body(a0, a1, dd, w1, b1, w2, b2, out):
    a = a0[0] + a1[0]
    d = (dd[0:1, :] + dd[1:2, :]).reshape(a.shape[0], 1)
    a = a / jnp.maximum(d, 1.0)
    h = jnp.dot(a, w1[...], preferred_element_type=jnp.float32) + b1[...]
    h = jnp.maximum(h, 0.0)
    out[...] = jnp.dot(h, w2[...], preferred_element_type=jnp.float32) + b2[...]


def kernel(x, edge_index, W1, b1, W2, b2):
    ei = edge_index.astype(jnp.int32)
    src, dst = pl.pallas_call(
        _prep_body,
        grid=(E_PAD // PREP_B,),
        in_specs=[pl.BlockSpec((2, PREP_B),
                               lambda g: (0, jnp.minimum(g, N_EDGES // PREP_B)))],
        out_specs=[pl.BlockSpec((PREP_B,), lambda g: (g,)),
                   pl.BlockSpec((PREP_B,), lambda g: (g,))],
        out_shape=[jax.ShapeDtypeStruct((E_PAD,), jnp.int32),
                   jax.ShapeDtypeStruct((E_PAD,), jnp.int32)],
    )(ei)
    zeros = jnp.zeros((RPT, D_FEAT), jnp.float32)
    zeros1 = jnp.zeros((ROWS_PAD,), jnp.float32)

    mesh = plsc.VectorSubcoreMesh(core_axis_name="c", subcore_axis_name="s",
                                  num_cores=NC, num_subcores=NS)
    sc = pl.kernel(
        _sc_body,
        out_type=(
            jax.ShapeDtypeStruct((NC, ROWS_PAD, D_FEAT), jnp.float32),
            jax.ShapeDtypeStruct((NC, ROWS_PAD), jnp.float32),
        ),
        mesh=mesh,
        scratch_types=[
            pltpu.VMEM((EPH,), jnp.int32),
            pltpu.VMEM((EPH,), jnp.int32),
            pltpu.VMEM((CHUNK,), jnp.int32),
            pltpu.VMEM((CHUNK,), jnp.int32),
            pltpu.VMEM((CHUNK,), jnp.int32),
            pltpu.VMEM((CHUNK, D_FEAT), jnp.float32),
            pltpu.VMEM((CHUNK, D_FEAT), jnp.float32),
            pltpu.VMEM((CHUNK, D_FEAT), jnp.float32),
            pltpu.VMEM((CHUNK,), jnp.float32),
            pltpu.VMEM_SHARED((ROWS_PAD, D_FEAT), jnp.float32),
            pltpu.VMEM_SHARED((ROWS_PAD,), jnp.float32),
        ] + [pltpu.SemaphoreType.DMA] * 9,
    )
    aggp, degp = sc(x, src, dst, zeros, zeros1)

    R = 1024
    grid = (-(-N_NODES // R),)
    out = pl.pallas_call(
        _mlp_body,
        grid=grid,
        in_specs=[
            pl.BlockSpec((1, R, D_FEAT), lambda i: (0, i, 0)),
            pl.BlockSpec((1, R, D_FEAT), lambda i: (1, i, 0)),
            pl.BlockSpec((2, R), lambda i: (0, i)),
            pl.BlockSpec((D_FEAT, D_HID), lambda i: (0, 0)),
            pl.BlockSpec((1, D_HID), lambda i: (0, 0)),
            pl.BlockSpec((D_HID, D_OUT), lambda i: (0, 0)),
            pl.BlockSpec((1, D_OUT), lambda i: (0, 0)),
        ],
        out_specs=pl.BlockSpec((R, D_OUT), lambda i: (i, 0)),
        out_shape=jax.ShapeDtypeStruct((N_NODES, D_OUT), jnp.float32),
    )(aggp, aggp, degp, W1, b1.reshape(1, D_HID), W2,
      b2.reshape(1, D_OUT))
    return out


# speedup vs baseline: 3.7075x; 1.1173x over previous
import jax
import jax.numpy as jnp
from jax import lax
from jax.experimental import pallas as pl
from jax.experimental.pallas import tpu as pltpu
from jax.experimental.pallas import tpu_sc as plsc

N_NODES = 10000
N_EDGES = 320000
D_FEAT = 128
D_HID = 256
D_OUT = 256

NC = 2
NS = 16
NW = NC * NS
CHUNK = 64
CPT = 168
EPT = CPT * CHUNK
E_PAD = NW * EPT
HALF = CPT // 2
EPH = HALF * CHUNK
ROWS_PAD = 10240
RPT = ROWS_PAD // NS


PREP_B = 24576


def _prep_body(ei, src_out, dst_out):
    g = pl.program_id(0)
    gi = g * PREP_B + jax.lax.broadcasted_iota(jnp.int32, (1, PREP_B), 1)
    real = gi < N_EDGES
    src = jnp.where(real, ei[0:1, :], gi & 8191)
    dst = jnp.where(real, ei[1:2, :], N_NODES + (gi & 127))
    src_out[...] = src.reshape(PREP_B)
    dst_out[...] = dst.reshape(PREP_B)


def _sc_body(x_hbm, src_hbm, dst_hbm, zeros_hbm, zeros1_hbm,
             aggp_hbm, degp_hbm,
             src_v, dst_v, db0, db1, db2, rows_v0, rows_v1, rows_v2, ones_v,
             agg_sh, deg_sh,
             sem_g0, sem_g1, sem_g2, sem_s0, sem_s1, sem_s2,
             sem_d0, sem_d1, sem_d2):
    c = lax.axis_index("c")
    s = lax.axis_index("s")
    wid = s * NC + c

    pltpu.sync_copy(zeros_hbm, agg_sh.at[pl.ds(s * RPT, RPT)])
    pltpu.sync_copy(zeros1_hbm.at[pl.ds(s * RPT, RPT)],
                    deg_sh.at[pl.ds(s * RPT, RPT)])

    ones = jnp.ones((16,), jnp.float32)
    for k in range(CHUNK // 16):
        ones_v[pl.ds(k * 16, 16)] = ones

    rows = (rows_v0, rows_v1, rows_v2)
    dstbuf = (db0, db1, db2)
    sem_g = (sem_g0, sem_g1, sem_g2)
    sem_s = (sem_s0, sem_s1, sem_s2)
    sem_d = (sem_d0, sem_d1, sem_d2)
    base = wid * EPT

    def gather(i, b):
        return pltpu.make_async_copy(
            x_hbm.at[src_v.at[pl.ds(i * CHUNK, CHUNK)]], rows[b], sem_g[b])

    def stage_dst(i, b):
        for j in range(CHUNK // 16):
            dstbuf[b][pl.ds(j * 16, 16)] = dst_v[pl.ds(i * CHUNK + j * 16, 16)]

    def drains(b):
        pltpu.make_async_copy(rows[b], agg_sh.at[dstbuf[b]], sem_s[b]).wait()
        pltpu.make_async_copy(ones_v, deg_sh.at[dstbuf[b]], sem_d[b]).wait()

    plsc.subcore_barrier()

    for h in range(2):
        pltpu.sync_copy(src_hbm.at[pl.ds(base + h * EPH, EPH)], src_v)
        pltpu.sync_copy(dst_hbm.at[pl.ds(base + h * EPH, EPH)], dst_v)

        stage_dst(0, 0)
        stage_dst(1, 1)
        gather(0, 0).start()
        gather(1, 1).start()

        def step(k, carry):
            for b in range(3):
                i = k * 3 + b
                gather(i, b).wait()
                pltpu.async_copy(rows[b], agg_sh.at[dstbuf[b]], sem_s[b],
                                 add=True)
                pltpu.async_copy(ones_v, deg_sh.at[dstbuf[b]], sem_d[b],
                                 add=True)
                nxt = i + 2
                slot = (b + 2) % 3

                @pl.when(nxt < HALF)
                def _():
                    @pl.when(i > 0)
                    def _():
                        drains(slot)
                    stage_dst(nxt, slot)
                    gather(nxt, slot).start()
            return carry

        lax.fori_loop(0, HALF // 3, step, 0)

        for b in range(3):
            drains(b)

    plsc.subcore_barrier()

    pltpu.sync_copy(agg_sh.at[pl.ds(s * RPT, RPT)],
                    aggp_hbm.at[c].at[pl.ds(s * RPT, RPT)])
    pltpu.sync_copy(deg_sh.at[pl.ds(s * RPT, RPT)],
                    degp_hbm.at[c].at[pl.ds(s * RPT, RPT)])


def _mlp_body(a0, a1, dd, w1, b1, w2, b2, out):
    a = a0[0] + a1[0]
    d = (dd[0:1, :] + dd[1:2, :]).reshape(a.shape[0], 1)
    a = a / jnp.maximum(d, 1.0)
    h = jnp.dot(a, w1[...], preferred_element_type=jnp.float32) + b1[...]
    h = jnp.maximum(h, 0.0)
    out[...] = jnp.dot(h, w2[...], preferred_element_type=jnp.float32) + b2[...]


def kernel(x, edge_index, W1, b1, W2, b2):
    ei = edge_index.astype(jnp.int32)
    src, dst = pl.pallas_call(
        _prep_body,
        grid=(E_PAD // PREP_B,),
        in_specs=[pl.BlockSpec((2, PREP_B),
                               lambda g: (0, jnp.minimum(g, N_EDGES // PREP_B)))],
        out_specs=[pl.BlockSpec((PREP_B,), lambda g: (g,)),
                   pl.BlockSpec((PREP_B,), lambda g: (g,))],
        out_shape=[jax.ShapeDtypeStruct((E_PAD,), jnp.int32),
                   jax.ShapeDtypeStruct((E_PAD,), jnp.int32)],
    )(ei)
    zeros = jnp.zeros((RPT, D_FEAT), jnp.float32)
    zeros1 = jnp.zeros((ROWS_PAD,), jnp.float32)

    mesh = plsc.VectorSubcoreMesh(core_axis_name="c", subcore_axis_name="s",
                                  num_cores=NC, num_subcores=NS)
    sc = pl.kernel(
        _sc_body,
        out_type=(
            jax.ShapeDtypeStruct((NC, ROWS_PAD, D_FEAT), jnp.float32),
            jax.ShapeDtypeStruct((NC, ROWS_PAD), jnp.float32),
        ),
        mesh=mesh,
        scratch_types=[
            pltpu.VMEM((EPH,), jnp.int32),
            pltpu.VMEM((EPH,), jnp.int32),
            pltpu.VMEM((CHUNK,), jnp.int32),
            pltpu.VMEM((CHUNK,), jnp.int32),
            pltpu.VMEM((CHUNK,), jnp.int32),
            pltpu.VMEM((CHUNK, D_FEAT), jnp.float32),
            pltpu.VMEM((CHUNK, D_FEAT), jnp.float32),
            pltpu.VMEM((CHUNK, D_FEAT), jnp.float32),
            pltpu.VMEM((CHUNK,), jnp.float32),
            pltpu.VMEM_SHARED((ROWS_PAD, D_FEAT), jnp.float32),
            pltpu.VMEM_SHARED((ROWS_PAD,), jnp.float32),
        ] + [pltpu.SemaphoreType.DMA] * 9,
    )
    aggp, degp = sc(x, src, dst, zeros, zeros1)

    R = 1024
    grid = (-(-N_NODES // R),)
    out = pl.pallas_call(
        _mlp_body,
        grid=grid,
        in_specs=[
            pl.BlockSpec((1, R, D_FEAT), lambda i: (0, i, 0)),
            pl.BlockSpec((1, R, D_FEAT), lambda i: (1, i, 0)),
            pl.BlockSpec((2, R), lambda i: (0, i)),
            pl.BlockSpec((D_FEAT, D_HID), lambda i: (0, 0)),
            pl.BlockSpec((1, D_HID), lambda i: (0, 0)),
            pl.BlockSpec((D_HID, D_OUT), lambda i: (0, 0)),
            pl.BlockSpec((1, D_OUT), lambda i: (0, 0)),
        ],
        out_specs=pl.BlockSpec((R, D_OUT), lambda i: (i, 0)),
        out_shape=jax.ShapeDtypeStruct((N_NODES, D_OUT), jnp.float32),
    )(aggp, aggp, degp, W1, b1.reshape(1, D_HID), W2,
      b2.reshape(1, D_OUT))
    return out
